# Initial kernel scaffold; baseline (speedup 1.0000x reference)
import jax
import jax.numpy as jnp
from jax.experimental import pallas as pl


def kernel(xyz, feature):
    raise NotImplementedError("write your pallas kernel here")


# speedup vs baseline: 18.9851x; 18.9851x over previous
import functools

import jax
import jax.numpy as jnp
from jax import lax
from jax.experimental import pallas as pl
from jax.experimental.pallas import tpu as pltpu
from jax.experimental.pallas import tpu_sc as plsc

B, N, C = 8, 2048, 64
NS = 16
NSAMPLE = 20
R2 = 256.0
L = 16
NWORKERS = 32
TILES_PER_B = NWORKERS // B
NQ = N // TILES_PER_B
LIST_LEN = 40
CCHUNK = 16


def _body(xyz_hbm, feat_hbm, oxyz_hbm, ofeat_hbm,
          xyz_tab, ftab, ids_ref, list_ref, obuf):
    cid = lax.axis_index("c")
    sid = lax.axis_index("s")
    wid = sid * 2 + cid
    b = wid // TILES_PER_B
    i0 = (wid % TILES_PER_B) * NQ

    iota = lax.iota(jnp.int32, L)
    zeros = jnp.zeros((L,), jnp.int32)
    dil_pos = iota + 1 + (iota >> 2)

    pltpu.sync_copy(xyz_hbm.at[b], xyz_tab)

    def one_query(i, carry):
        qi = jnp.full((L,), i0 + i, jnp.int32)
        qx = plsc.load_gather(xyz_tab, [qi, zeros])
        qy = plsc.load_gather(xyz_tab, [qi, zeros + 1])
        qz = plsc.load_gather(xyz_tab, [qi, zeros + 2])

        def cond(c):
            j, cnt = c
            return (cnt < NSAMPLE) & (j < N)

        def scan_chunk(c):
            j, cnt = c
            jv = iota + j
            xj = plsc.load_gather(xyz_tab, [jv, zeros])
            yj = plsc.load_gather(xyz_tab, [jv, zeros + 1])
            zj = plsc.load_gather(xyz_tab, [jv, zeros + 2])
            dx = xj - qx
            dy = yj - qy
            dz = zj - qz
            d2 = dx * dx + dy * dy + dz * dz
            m = d2 < R2
            mi = m.astype(jnp.int32)
            inc = plsc.cumsum(mi)
            pos = jnp.full((L,), cnt, jnp.int32) + inc - mi
            plsc.store_scatter(list_ref, [pos], jv, mask=m)
            return j + L, cnt + jnp.sum(mi)

        _, cnt = lax.while_loop(cond, scan_chunk, (jnp.int32(0), jnp.int32(0)))
        hits = plsc.load_gather(list_ref, [dil_pos])
        first = plsc.load_gather(list_ref, [zeros])
        ids = jnp.where(dil_pos < jnp.full((L,), cnt, jnp.int32), hits, first)
        ids_ref[i, :] = ids
        return carry

    lax.fori_loop(0, NQ, one_query, 0)

    def xyz_row(i, carry):
        idxv = ids_ref[i, :]
        for c in range(3):
            obuf[c, i, :] = plsc.load_gather(xyz_tab, [idxv, zeros + c])
        return carry

    lax.fori_loop(0, NQ, xyz_row, 0)
    pltpu.sync_copy(obuf.at[pl.ds(0, 3)], oxyz_hbm.at[b, :, pl.ds(i0, NQ), :])

    for cc in range(C // CCHUNK):
        pltpu.sync_copy(feat_hbm.at[b, :, pl.ds(cc * CCHUNK, CCHUNK)], ftab)
        for half in range(2):
            def feat_row(i, carry, half=half):
                idxv = ids_ref[i, :]
                for c8 in range(8):
                    obuf[c8, i, :] = plsc.load_gather(
                        ftab, [idxv, zeros + (half * 8 + c8)])
                return carry

            lax.fori_loop(0, NQ, feat_row, 0)
            pltpu.sync_copy(
                obuf,
                ofeat_hbm.at[b, pl.ds(cc * CCHUNK + half * 8, 8),
                             pl.ds(i0, NQ), :])


@functools.cache
def _sc_call():
    return pl.kernel(
        _body,
        out_type=(
            jax.ShapeDtypeStruct((B, 3, N, NS), jnp.float32),
            jax.ShapeDtypeStruct((B, C, N, NS), jnp.float32),
        ),
        mesh=plsc.VectorSubcoreMesh(core_axis_name="c", subcore_axis_name="s",
                                    num_cores=2, num_subcores=16),
        scratch_types=[
            pltpu.VMEM((N, 3), jnp.float32),
            pltpu.VMEM((N, CCHUNK), jnp.float32),
            pltpu.VMEM((NQ, NS), jnp.int32),
            pltpu.VMEM((LIST_LEN,), jnp.int32),
            pltpu.VMEM((8, NQ, NS), jnp.float32),
        ],
        compiler_params=pltpu.CompilerParams(use_tc_tiling_on_sc=False,
                                             needs_layout_passes=False),
    )


@jax.jit
def kernel(xyz, feature):
    return _sc_call()(xyz, feature)


# speedup vs baseline: 27.1303x; 1.4290x over previous
import functools

import jax
import jax.numpy as jnp
from jax import lax
from jax.experimental import pallas as pl
from jax.experimental.pallas import tpu as pltpu
from jax.experimental.pallas import tpu_sc as plsc

B, N, C = 8, 2048, 64
NS = 16
NSAMPLE = 20
R2 = 256.0
L = 16
NWORKERS = 32
TILES_PER_B = NWORKERS // B
NQ = N // TILES_PER_B
CCHUNK = 16


def _body(xyzt_hbm, feat_hbm, oxyz_hbm, ofeat_hbm,
          soa, ftab, ids_ref, list_ref, cnt_ref, obuf):
    cid = lax.axis_index("c")
    sid = lax.axis_index("s")
    wid = sid * 2 + cid
    b = wid // TILES_PER_B
    i0 = (wid % TILES_PER_B) * NQ

    iota = lax.iota(jnp.int32, L)
    zeros = jnp.zeros((L,), jnp.int32)
    dil_pos = iota + 1 + (iota >> 2)

    pltpu.sync_copy(xyzt_hbm.at[b], soa)

    x0 = soa[0, pl.ds(0, L)]
    y0 = soa[1, pl.ds(0, L)]
    z0 = soa[2, pl.ds(0, L)]
    x1 = soa[0, pl.ds(L, L)]
    y1 = soa[1, pl.ds(L, L)]
    z1 = soa[2, pl.ds(L, L)]

    @plsc.parallel_loop(0, NQ, unroll=2)
    def _pass_a(i):
        spl_i = jnp.full((L,), i, jnp.int32)
        qi = jnp.full((L,), i0 + i, jnp.int32)
        qx = plsc.load_gather(soa, [zeros, qi])
        qy = plsc.load_gather(soa, [zeros + 1, qi])
        qz = plsc.load_gather(soa, [zeros + 2, qi])

        dx0 = x0 - qx
        dy0 = y0 - qy
        dz0 = z0 - qz
        m0 = dx0 * dx0 + dy0 * dy0 + dz0 * dz0 < R2
        mi0 = m0.astype(jnp.int32)
        inc0 = plsc.cumsum(mi0)
        pos0 = inc0 - mi0
        plsc.store_scatter(list_ref, [spl_i, pos0], iota,
                           mask=m0 & (pos0 < NSAMPLE))
        c0 = jnp.sum(mi0)

        dx1 = x1 - qx
        dy1 = y1 - qy
        dz1 = z1 - qz
        m1 = dx1 * dx1 + dy1 * dy1 + dz1 * dz1 < R2
        mi1 = m1.astype(jnp.int32)
        inc1 = plsc.cumsum(mi1)
        pos1 = jnp.full((L,), c0, jnp.int32) + inc1 - mi1
        plsc.store_scatter(list_ref, [spl_i, pos1], iota + L,
                           mask=m1 & (pos1 < NSAMPLE))
        cnt = c0 + jnp.sum(mi1)

        cnt_ref[i] = cnt
        cntv = jnp.full((L,), cnt, jnp.int32)
        hits = plsc.load_gather(list_ref, [spl_i, dil_pos])
        first = plsc.load_gather(list_ref, [spl_i, zeros])
        ids_ref[i, :] = jnp.where(dil_pos < cntv, hits, first)

    def _fixup(i, carry):
        @pl.when(cnt_ref[i] < NSAMPLE)
        def _():
            spl_i = jnp.full((L,), i, jnp.int32)
            qi = jnp.full((L,), i0 + i, jnp.int32)
            qx = plsc.load_gather(soa, [zeros, qi])
            qy = plsc.load_gather(soa, [zeros + 1, qi])
            qz = plsc.load_gather(soa, [zeros + 2, qi])

            def cond(c):
                j, cnt = c
                return (cnt < NSAMPLE) & (j < N)

            def step(c):
                j, cnt = c
                xj = soa[0, pl.ds(j, L)]
                yj = soa[1, pl.ds(j, L)]
                zj = soa[2, pl.ds(j, L)]
                dx = xj - qx
                dy = yj - qy
                dz = zj - qz
                m = dx * dx + dy * dy + dz * dz < R2
                mi = m.astype(jnp.int32)
                inc = plsc.cumsum(mi)
                pos = jnp.full((L,), cnt, jnp.int32) + inc - mi
                plsc.store_scatter(list_ref, [spl_i, pos], iota + j,
                                   mask=m & (pos < NSAMPLE))
                return j + L, cnt + jnp.sum(mi)

            _, cnt = lax.while_loop(cond, step,
                                    (jnp.int32(0), jnp.int32(0)))
            cntv = jnp.full((L,), cnt, jnp.int32)
            hits = plsc.load_gather(list_ref, [spl_i, dil_pos])
            first = plsc.load_gather(list_ref, [spl_i, zeros])
            ids_ref[i, :] = jnp.where(dil_pos < cntv, hits, first)
        return carry

    lax.fori_loop(0, NQ, _fixup, 0)

    @plsc.parallel_loop(0, NQ, unroll=4)
    def _xyz_row(i):
        idxv = ids_ref[i, :]
        vals = [plsc.load_gather(soa, [zeros + c, idxv]) for c in range(3)]
        for c in range(3):
            obuf[c, i, :] = vals[c]

    pltpu.sync_copy(obuf.at[pl.ds(0, 3)], oxyz_hbm.at[b, :, pl.ds(i0, NQ), :])

    for cc in range(C // CCHUNK):
        pltpu.sync_copy(feat_hbm.at[b, :, pl.ds(cc * CCHUNK, CCHUNK)], ftab)
        for half in range(2):
            @plsc.parallel_loop(0, NQ, unroll=4)
            def _feat_row(i, half=half):
                idxv = ids_ref[i, :]
                vals = [plsc.load_gather(ftab, [idxv, zeros + (half * 8 + c8)])
                        for c8 in range(8)]
                for c8 in range(8):
                    obuf[c8, i, :] = vals[c8]

            pltpu.sync_copy(
                obuf,
                ofeat_hbm.at[b, pl.ds(cc * CCHUNK + half * 8, 8),
                             pl.ds(i0, NQ), :])


@functools.cache
def _sc_call():
    return pl.kernel(
        _body,
        out_type=(
            jax.ShapeDtypeStruct((B, 3, N, NS), jnp.float32),
            jax.ShapeDtypeStruct((B, C, N, NS), jnp.float32),
        ),
        mesh=plsc.VectorSubcoreMesh(core_axis_name="c", subcore_axis_name="s",
                                    num_cores=2, num_subcores=16),
        scratch_types=[
            pltpu.VMEM((3, N), jnp.float32),
            pltpu.VMEM((N, CCHUNK), jnp.float32),
            pltpu.VMEM((NQ, NS), jnp.int32),
            pltpu.VMEM((NQ, NSAMPLE), jnp.int32),
            pltpu.SMEM((NQ,), jnp.int32),
            pltpu.VMEM((8, NQ, NS), jnp.float32),
        ],
        compiler_params=pltpu.CompilerParams(use_tc_tiling_on_sc=False,
                                             needs_layout_passes=False),
    )


@jax.jit
def kernel(xyz, feature):
    xyzt = jnp.transpose(xyz, (0, 2, 1))
    return _sc_call()(xyzt, feature)


# speedup vs baseline: 45.7429x; 1.6860x over previous
import functools

import jax
import jax.numpy as jnp
from jax import lax
from jax.experimental import pallas as pl
from jax.experimental.pallas import tpu as pltpu
from jax.experimental.pallas import tpu_sc as plsc

B, N, C = 8, 2048, 64
NS = 16
NSAMPLE = 20
R2 = 256.0
L = 16
NWORKERS = 32
TILES_PER_B = NWORKERS // B
NQ = N // TILES_PER_B
CCHUNK = 16


def _body(xyzt_hbm, feat_hbm, oxyz_hbm, ofeat_hbm,
          soa, ftab, ids_ref, list_ref, cnt_ref, obuf, sem, osem):
    cid = lax.axis_index("c")
    sid = lax.axis_index("s")
    wid = sid * 2 + cid
    b = wid // TILES_PER_B
    i0 = (wid % TILES_PER_B) * NQ

    iota = lax.iota(jnp.int32, L)
    zeros = jnp.zeros((L,), jnp.int32)
    dil_pos = iota + 1 + (iota >> 2)

    pltpu.sync_copy(xyzt_hbm.at[b], soa)
    ftab_cp = pltpu.async_copy(feat_hbm.at[b, :, pl.ds(0, CCHUNK)], ftab, sem)

    x0 = soa[0, pl.ds(0, L)]
    y0 = soa[1, pl.ds(0, L)]
    z0 = soa[2, pl.ds(0, L)]
    x1 = soa[0, pl.ds(L, L)]
    y1 = soa[1, pl.ds(L, L)]
    z1 = soa[2, pl.ds(L, L)]

    with jax.named_scope("ball_query_a"):
        @plsc.parallel_loop(0, NQ, unroll=2)
        def _pass_a(i):
            spl_i = jnp.full((L,), i, jnp.int32)
            qi = jnp.full((L,), i0 + i, jnp.int32)
            qx = plsc.load_gather(soa, [zeros, qi])
            qy = plsc.load_gather(soa, [zeros + 1, qi])
            qz = plsc.load_gather(soa, [zeros + 2, qi])

            dx0 = x0 - qx
            dy0 = y0 - qy
            dz0 = z0 - qz
            m0 = dx0 * dx0 + dy0 * dy0 + dz0 * dz0 < R2
            mi0 = m0.astype(jnp.int32)
            inc0 = plsc.cumsum(mi0)
            pos0 = inc0 - mi0
            plsc.store_scatter(list_ref, [spl_i, pos0], iota,
                               mask=m0 & (pos0 < NSAMPLE))
            c0 = jnp.sum(mi0)

            dx1 = x1 - qx
            dy1 = y1 - qy
            dz1 = z1 - qz
            m1 = dx1 * dx1 + dy1 * dy1 + dz1 * dz1 < R2
            mi1 = m1.astype(jnp.int32)
            inc1 = plsc.cumsum(mi1)
            pos1 = jnp.full((L,), c0, jnp.int32) + inc1 - mi1
            plsc.store_scatter(list_ref, [spl_i, pos1], iota + L,
                               mask=m1 & (pos1 < NSAMPLE))
            cnt = c0 + jnp.sum(mi1)

            cnt_ref[i] = cnt
            cntv = jnp.full((L,), cnt, jnp.int32)
            hits = plsc.load_gather(list_ref, [spl_i, dil_pos])
            first = plsc.load_gather(list_ref, [spl_i, zeros])
            ids_ref[i, :] = jnp.where(dil_pos < cntv, hits, first)

    with jax.named_scope("ball_query_fixup"):
        def _fixup(i, carry):
            @pl.when(cnt_ref[i] < NSAMPLE)
            def _():
                spl_i = jnp.full((L,), i, jnp.int32)
                qi = jnp.full((L,), i0 + i, jnp.int32)
                qx = plsc.load_gather(soa, [zeros, qi])
                qy = plsc.load_gather(soa, [zeros + 1, qi])
                qz = plsc.load_gather(soa, [zeros + 2, qi])

                def cond(c):
                    j, cnt = c
                    return (cnt < NSAMPLE) & (j < N)

                def step(c):
                    j, cnt = c
                    xj = soa[0, pl.ds(j, L)]
                    yj = soa[1, pl.ds(j, L)]
                    zj = soa[2, pl.ds(j, L)]
                    dx = xj - qx
                    dy = yj - qy
                    dz = zj - qz
                    m = dx * dx + dy * dy + dz * dz < R2
                    mi = m.astype(jnp.int32)
                    inc = plsc.cumsum(mi)
                    pos = jnp.full((L,), cnt, jnp.int32) + inc - mi
                    plsc.store_scatter(list_ref, [spl_i, pos], iota + j,
                                       mask=m & (pos < NSAMPLE))
                    return j + L, cnt + jnp.sum(mi)

                _, cnt = lax.while_loop(cond, step,
                                        (jnp.int32(0), jnp.int32(0)))
                cntv = jnp.full((L,), cnt, jnp.int32)
                hits = plsc.load_gather(list_ref, [spl_i, dil_pos])
                first = plsc.load_gather(list_ref, [spl_i, zeros])
                ids_ref[i, :] = jnp.where(dil_pos < cntv, hits, first)
            return carry

        lax.fori_loop(0, NQ, _fixup, 0)

    with jax.named_scope("xyz_gather"):
        @plsc.parallel_loop(0, NQ, unroll=4)
        def _xyz_row(i):
            idxv = ids_ref[i, :]
            vals = [plsc.load_gather(soa, [zeros + c, idxv]) for c in range(3)]
            for c in range(3):
                obuf[c, pl.ds(i * NS, NS)] = vals[c]

        prev_cp = pltpu.async_copy(
            obuf.at[pl.ds(0, 3)],
            oxyz_hbm.at[b, :, pl.ds(i0 * NS, NQ * NS)], osem)

    with jax.named_scope("feat_gather"):
        ftab_cp.wait()
        pending = {0: prev_cp, 4: None}
        for q in range(C // 4):
            cc, within = divmod(q, 4)
            r0 = 4 * ((q + 1) % 2)
            if within == 0 and cc > 0:
                pltpu.sync_copy(feat_hbm.at[b, :, pl.ds(cc * CCHUNK, CCHUNK)],
                                ftab)
            if pending[r0] is not None:
                pending[r0].wait()

            @plsc.parallel_loop(0, NQ, unroll=4)
            def _feat_row(i, within=within, r0=r0):
                idxv = ids_ref[i, :]
                vals = [plsc.load_gather(ftab,
                                         [idxv, zeros + (within * 4 + c4)])
                        for c4 in range(4)]
                for c4 in range(4):
                    obuf[r0 + c4, pl.ds(i * NS, NS)] = vals[c4]

            pending[r0] = pltpu.async_copy(
                obuf.at[pl.ds(r0, 4)],
                ofeat_hbm.at[b, pl.ds(cc * CCHUNK + within * 4, 4),
                             pl.ds(i0 * NS, NQ * NS)], osem)
        pending[0].wait()
        pending[4].wait()


@functools.cache
def _sc_call():
    return pl.kernel(
        _body,
        out_type=(
            jax.ShapeDtypeStruct((B, 3, N * NS), jnp.float32),
            jax.ShapeDtypeStruct((B, C, N * NS), jnp.float32),
        ),
        mesh=plsc.VectorSubcoreMesh(core_axis_name="c", subcore_axis_name="s",
                                    num_cores=2, num_subcores=16),
        scratch_types=[
            pltpu.VMEM((3, N), jnp.float32),
            pltpu.VMEM((N, CCHUNK), jnp.float32),
            pltpu.VMEM((NQ, NS), jnp.int32),
            pltpu.VMEM((NQ, NSAMPLE), jnp.int32),
            pltpu.SMEM((NQ,), jnp.int32),
            pltpu.VMEM((8, NQ * NS), jnp.float32),
            pltpu.SemaphoreType.DMA,
            pltpu.SemaphoreType.DMA,
        ],
        compiler_params=pltpu.CompilerParams(use_tc_tiling_on_sc=False,
                                             needs_layout_passes=False),
    )


@jax.jit
def kernel(xyz, feature):
    xyzt = jnp.transpose(xyz, (0, 2, 1))
    oxyz, ofeat = _sc_call()(xyzt, feature)
    return (oxyz.reshape(B, 3, N, NS), ofeat.reshape(B, C, N, NS))


# speedup vs baseline: 153.0982x; 3.3469x over previous
import functools

import jax
import jax.numpy as jnp
from jax import lax
from jax.experimental import pallas as pl
from jax.experimental.pallas import tpu as pltpu
from jax.experimental.pallas import tpu_sc as plsc

B, N, C = 8, 2048, 64
NS = 16
NSAMPLE = 20
R2 = 256.0
L = 16
NWORKERS = 32
TILES_PER_B = NWORKERS // B
NQ = N // TILES_PER_B
CCHUNK = 16


def _body(xyzt_hbm, feat_hbm, oxyz_hbm, ofeat_hbm,
          soa, ftab, ids_ref, list_ref, cnt_ref, obuf, sem, osem):
    cid = lax.axis_index("c")
    sid = lax.axis_index("s")
    wid = sid * 2 + cid
    b = wid // TILES_PER_B
    i0 = (wid % TILES_PER_B) * NQ

    iota = lax.iota(jnp.int32, L)
    zeros = jnp.zeros((L,), jnp.int32)
    dil_pos = iota + 1 + (iota >> 2)

    pltpu.sync_copy(xyzt_hbm.at[b], soa)
    ftab_cp = pltpu.async_copy(feat_hbm.at[b, :, pl.ds(0, CCHUNK)], ftab, sem)

    x0 = soa[0, pl.ds(0, L)]
    y0 = soa[1, pl.ds(0, L)]
    z0 = soa[2, pl.ds(0, L)]
    x1 = soa[0, pl.ds(L, L)]
    y1 = soa[1, pl.ds(L, L)]
    z1 = soa[2, pl.ds(L, L)]

    with jax.named_scope("ball_query_a"):
        @plsc.parallel_loop(0, NQ, unroll=2)
        def _pass_a(i):
            spl_i = jnp.full((L,), i, jnp.int32)
            qi = jnp.full((L,), i0 + i, jnp.int32)
            qx = plsc.load_gather(soa, [zeros, qi])
            qy = plsc.load_gather(soa, [zeros + 1, qi])
            qz = plsc.load_gather(soa, [zeros + 2, qi])

            dx0 = x0 - qx
            dy0 = y0 - qy
            dz0 = z0 - qz
            m0 = dx0 * dx0 + dy0 * dy0 + dz0 * dz0 < R2
            mi0 = m0.astype(jnp.int32)
            inc0 = plsc.cumsum(mi0)
            pos0 = inc0 - mi0
            plsc.store_scatter(list_ref, [spl_i, pos0], iota,
                               mask=m0 & (pos0 < NSAMPLE))
            c0 = jnp.sum(mi0)

            dx1 = x1 - qx
            dy1 = y1 - qy
            dz1 = z1 - qz
            m1 = dx1 * dx1 + dy1 * dy1 + dz1 * dz1 < R2
            mi1 = m1.astype(jnp.int32)
            inc1 = plsc.cumsum(mi1)
            pos1 = jnp.full((L,), c0, jnp.int32) + inc1 - mi1
            plsc.store_scatter(list_ref, [spl_i, pos1], iota + L,
                               mask=m1 & (pos1 < NSAMPLE))
            cnt = c0 + jnp.sum(mi1)

            cnt_ref[i] = cnt
            cntv = jnp.full((L,), cnt, jnp.int32)
            hits = plsc.load_gather(list_ref, [spl_i, dil_pos])
            first = plsc.load_gather(list_ref, [spl_i, zeros])
            ids = jnp.where(dil_pos < cntv, hits, first)
            plsc.store_scatter(ids_ref, [iota, spl_i], ids)

    with jax.named_scope("ball_query_fixup"):
        def _fixup(i, carry):
            @pl.when(cnt_ref[i] < NSAMPLE)
            def _():
                spl_i = jnp.full((L,), i, jnp.int32)
                qi = jnp.full((L,), i0 + i, jnp.int32)
                qx = plsc.load_gather(soa, [zeros, qi])
                qy = plsc.load_gather(soa, [zeros + 1, qi])
                qz = plsc.load_gather(soa, [zeros + 2, qi])

                def cond(c):
                    j, cnt = c
                    return (cnt < NSAMPLE) & (j < N)

                def step(c):
                    j, cnt = c
                    xj = soa[0, pl.ds(j, L)]
                    yj = soa[1, pl.ds(j, L)]
                    zj = soa[2, pl.ds(j, L)]
                    dx = xj - qx
                    dy = yj - qy
                    dz = zj - qz
                    m = dx * dx + dy * dy + dz * dz < R2
                    mi = m.astype(jnp.int32)
                    inc = plsc.cumsum(mi)
                    pos = jnp.full((L,), cnt, jnp.int32) + inc - mi
                    plsc.store_scatter(list_ref, [spl_i, pos], iota + j,
                                       mask=m & (pos < NSAMPLE))
                    return j + L, cnt + jnp.sum(mi)

                _, cnt = lax.while_loop(cond, step,
                                        (jnp.int32(0), jnp.int32(0)))
                cntv = jnp.full((L,), cnt, jnp.int32)
                hits = plsc.load_gather(list_ref, [spl_i, dil_pos])
                first = plsc.load_gather(list_ref, [spl_i, zeros])
                ids = jnp.where(dil_pos < cntv, hits, first)
                plsc.store_scatter(ids_ref, [iota, spl_i], ids)
            return carry

        lax.fori_loop(0, NQ, _fixup, 0)

    with jax.named_scope("xyz_gather"):
        @plsc.parallel_loop(0, (NQ // L) * NS, unroll=4)
        def _xyz_blk(t):
            k = t // NS
            s = t % NS
            base = k * L
            idxc = ids_ref[s, pl.ds(base, L)]
            vals = [plsc.load_gather(soa, [zeros + c, idxc])
                    for c in range(3)]
            for c in range(3):
                obuf[c, s // 8, k // 8, s % 8, pl.ds((k % 8) * L, L)] = vals[c]

        prev_cp = pltpu.async_copy(
            obuf.at[pl.ds(0, 3)],
            oxyz_hbm.at[b, :, :, pl.ds(i0 // 128, NQ // 128), :, :], osem)

    with jax.named_scope("feat_gather"):
        ftab_cp.wait()
        pending = {0: prev_cp, 4: None}
        for q in range(C // 4):
            cc, within = divmod(q, 4)
            r0 = 4 * ((q + 1) % 2)
            if within == 0 and cc > 0:
                pltpu.sync_copy(feat_hbm.at[b, :, pl.ds(cc * CCHUNK, CCHUNK)],
                                ftab)
            if pending[r0] is not None:
                pending[r0].wait()

            @plsc.parallel_loop(0, (NQ // L) * NS, unroll=4)
            def _feat_blk(t, within=within, r0=r0):
                k = t // NS
                s = t % NS
                base = k * L
                idxc = ids_ref[s, pl.ds(base, L)]
                vals = [plsc.load_gather(
                            ftab, [idxc, zeros + (within * 4 + c4)])
                        for c4 in range(4)]
                for c4 in range(4):
                    obuf[r0 + c4, s // 8, k // 8, s % 8,
                         pl.ds((k % 8) * L, L)] = vals[c4]

            pending[r0] = pltpu.async_copy(
                obuf.at[pl.ds(r0, 4)],
                ofeat_hbm.at[b, pl.ds(cc * CCHUNK + within * 4, 4), :,
                             pl.ds(i0 // 128, NQ // 128), :, :], osem)
        pending[0].wait()
        pending[4].wait()


@functools.cache
def _sc_call():
    return pl.kernel(
        _body,
        out_type=(
            jax.ShapeDtypeStruct((B, 3, NS // 8, N // 128, 8, 128),
                                 jnp.float32),
            jax.ShapeDtypeStruct((B, C, NS // 8, N // 128, 8, 128),
                                 jnp.float32),
        ),
        mesh=plsc.VectorSubcoreMesh(core_axis_name="c", subcore_axis_name="s",
                                    num_cores=2, num_subcores=16),
        scratch_types=[
            pltpu.VMEM((3, N), jnp.float32),
            pltpu.VMEM((N, CCHUNK), jnp.float32),
            pltpu.VMEM((NS, NQ), jnp.int32),
            pltpu.VMEM((NQ, NSAMPLE), jnp.int32),
            pltpu.SMEM((NQ,), jnp.int32),
            pltpu.VMEM((8, NS // 8, NQ // 128, 8, 128),
                       jnp.float32),
            pltpu.SemaphoreType.DMA,
            pltpu.SemaphoreType.DMA,
        ],
        compiler_params=pltpu.CompilerParams(use_tc_tiling_on_sc=False,
                                             needs_layout_passes=False),
    )


@jax.jit
def kernel(xyz, feature):
    xyzt = jnp.transpose(xyz, (0, 2, 1))
    oxyz, ofeat = _sc_call()(xyzt, feature)
    oxyz = jnp.transpose(oxyz, (0, 1, 3, 5, 2, 4)).reshape(B, 3, N, NS)
    ofeat = jnp.transpose(ofeat, (0, 1, 3, 5, 2, 4)).reshape(B, C, N, NS)
    return (oxyz, ofeat)
